# baseline (speedup 1.0000x reference)
import functools

import jax
import jax.numpy as jnp
from jax import lax
from jax.experimental import pallas as pl
from jax.experimental.pallas import tpu as pltpu
from jax.experimental.pallas import tpu_sc as plsc

EPS = 1e-05

_B, _C, _H, _W = 8, 96, 224, 224
_HW = _H * _W
_ROW = 128
_BLK = 3584
_NB = _HW // _BLK

_NC, _NS = 2, 16
_NCHUNK = 4
_DCH = _HW // _NCHUNK
_DUMP = _DCH
_ACC_ROWS = _DCH + 16
_SCAN = 3200
_SCAN_LO = 3072
_K = 128
_KH = 64
_NHG = _SCAN // _KH
_TSLICE = _DCH // _NS
_ZR = 16


def _stage1_body(x_ref, w_ref, b_ref, xwT_ref, dest_ref, off_ref):
    jb = pl.program_id(1)
    x_blk = x_ref[0]
    oa = lax.dot(w_ref[...], x_blk,
                 preferred_element_type=jnp.float32) + b_ref[...]
    off_y = oa[0:1] * float(_H)
    off_x = oa[1:2] * float(_W)
    att = jnp.exp(oa[2:3])
    p = jb * _BLK + lax.broadcasted_iota(jnp.int32, (1, _BLK), 1)
    dy = (p // _W).astype(jnp.float32)
    dx = (p % _W).astype(jnp.float32)
    dest_y = jnp.clip(jnp.round(dy + off_y).astype(jnp.int32), 0, _H - 1)
    dest_x = jnp.clip(jnp.round(dx + off_x).astype(jnp.int32), 0, _W - 1)
    dest = dest_y * _W + dest_x
    xw_t = (x_blk * att).T
    att_t = att.T
    pad = jnp.zeros((_BLK, _ROW - _C - 1), jnp.float32)
    xwT_ref[0] = jnp.concatenate([xw_t, att_t, pad], axis=1)
    dest_ref[0, 0] = dest[0]
    off_ref[0] = jnp.concatenate([off_y, off_x], axis=0)


@functools.lru_cache(maxsize=1)
def _make_sc_scatter():
    mesh = plsc.VectorSubcoreMesh(core_axis_name="c", subcore_axis_name="s")
    return functools.partial(
        pl.kernel,
        mesh=mesh,
        out_type=jax.ShapeDtypeStruct((2 * _HW, _ROW), jnp.float32),
        scratch_types=[
            pltpu.VMEM((_SCAN,), jnp.int32),
            pltpu.VMEM((_NHG + 2, _KH), jnp.int32),
            pltpu.VMEM((_KH, _ROW), jnp.float32),
            pltpu.VMEM((_KH, _ROW), jnp.float32),
            pltpu.VMEM((_ZR, _ROW), jnp.float32),
            pltpu.VMEM_SHARED((_ACC_ROWS, _ROW), jnp.float32),
            pltpu.SemaphoreType.DMA,
            pltpu.SemaphoreType.DMA,
        ],
    )(_sc_scatter_body)


def _sc_scatter_body(xwT_hbm, dest_hbm, outraw_hbm,
                     destbuf, dst2d, bufA, bufB, zbuf, acc, semA, semB):
    cid = lax.axis_index("c")
    sid = lax.axis_index("s")
    zero16 = jnp.zeros((16,), jnp.float32)

    def zb_body(r, carry):
        for c in range(_ROW // 16):
            zbuf[r, pl.ds(c * 16, 16)] = zero16
        return carry

    lax.fori_loop(0, _ZR, zb_body, 0)

    base = sid * _TSLICE
    sbase = pl.multiple_of(
        sid * _SCAN_LO + jnp.minimum(sid, 8) * _K, _K)
    nhg = jnp.where(sid < 8, _SCAN // _KH, _SCAN_LO // _KH)
    lo0 = cid * 2 * _DCH
    lo1 = (cid * 2 + 1) * _DCH

    def batch_body(b, carry):
        pltpu.sync_copy(dest_hbm.at[b].at[pl.ds(sbase, _SCAN)],
                        destbuf)
        for kk, lo in ((0, lo0), (1, lo1)):
            for z in range(_TSLICE // _ZR):
                pltpu.sync_copy(zbuf, acc.at[pl.ds(base + z * _ZR, _ZR)])
            plsc.subcore_barrier()

            def build(r, carry2):
                for u in range(_KH // 16):
                    off = pl.multiple_of(r * _KH + u * 16, 16)
                    d16 = destbuf[pl.ds(off, 16)]
                    m = (d16 >= lo) & (d16 < lo + _DCH)
                    dst2d[r, pl.ds(u * 16, 16)] = jnp.where(
                        m, d16 - lo, _DUMP + sid)
                return carry2

            lax.fori_loop(0, nhg, build, 0)

            def gissue(hg, buf, sem):
                row0 = pl.multiple_of(b * _HW + sbase + hg * _KH, _KH)
                return pltpu.async_copy(
                    xwT_hbm.at[pl.ds(row0, _KH)], buf, sem)

            def gwait(hg, buf, sem):
                row0 = pl.multiple_of(b * _HW + sbase + hg * _KH, _KH)
                pltpu.make_async_copy(
                    xwT_hbm.at[pl.ds(row0, _KH)], buf, sem).wait()

            gissue(0, bufA, semA)

            def gs_body(j2, carry2):
                hga = j2 * 2
                hgb = hga + 1
                gwait(hga, bufA, semA)
                gissue(hgb, bufB, semB)
                pltpu.sync_copy(bufA, acc.at[dst2d.at[hga]], add=True)
                gwait(hgb, bufB, semB)

                @pl.when(hgb + 1 < nhg)
                def _():
                    gissue(hgb + 1, bufA, semA)

                pltpu.sync_copy(bufB, acc.at[dst2d.at[hgb]], add=True)
                return carry2

            lax.fori_loop(0, nhg // 2, gs_body, 0)
            plsc.subcore_barrier()
            pltpu.sync_copy(
                acc.at[pl.ds(base, _TSLICE)],
                outraw_hbm.at[pl.ds(b * _HW + lo + base, _TSLICE)])
        return carry

    lax.fori_loop(0, 2, batch_body, 0)


def _stage3_body(raw_ref, out_ref):
    raw = raw_ref[0]
    fa = raw[:, 0:_C]
    aa = raw[:, _C:_C + 1] + EPS
    out_ref[0] = (fa / aa).T


def kernel(x, W, b):
    B, C, H, Wd = x.shape
    xf = x.reshape(B, C, _HW)
    sc_call = _make_sc_scatter()
    wr = W
    br = b.reshape(3, 1)

    stage1 = pl.pallas_call(
        _stage1_body,
        grid=(2, _NB),
        in_specs=[
            pl.BlockSpec((1, C, _BLK), lambda i, j: (i, 0, j)),
            pl.BlockSpec((3, C), lambda i, j: (0, 0)),
            pl.BlockSpec((3, 1), lambda i, j: (0, 0)),
        ],
        out_specs=[
            pl.BlockSpec((1, _BLK, _ROW), lambda i, j: (i, j, 0)),
            pl.BlockSpec((1, 1, _BLK), lambda i, j: (i, 0, j)),
            pl.BlockSpec((1, 2, _BLK), lambda i, j: (i, 0, j)),
        ],
        out_shape=[
            jax.ShapeDtypeStruct((2, _HW, _ROW), jnp.float32),
            jax.ShapeDtypeStruct((2, 1, _HW), jnp.int32),
            jax.ShapeDtypeStruct((2, 2, _HW), jnp.float32),
        ],
    )
    stage3 = pl.pallas_call(
        _stage3_body,
        grid=(2, _NB),
        in_specs=[pl.BlockSpec((1, _BLK, _ROW), lambda i, j: (i, j, 0))],
        out_specs=pl.BlockSpec((1, C, _BLK), lambda i, j: (i, 0, j)),
        out_shape=jax.ShapeDtypeStruct((2, C, _HW), jnp.float32),
    )

    xwTs, dests, offs = [], [], []
    for bi in range(0, B, 2):
        xwT, dest, off = stage1(xf[bi:bi + 2], wr, br)
        xwTs.append(xwT.reshape(2 * _HW, _ROW))
        dests.append(jnp.pad(dest.reshape(2, _HW), ((0, 0), (0, _K)),
                             constant_values=-1))
        offs.append(off)
    nP = B // 2
    raws = [sc_call(xwTs[i], dests[i]) for i in range(nP)]
    outs = [stage3(raws[i].reshape(2, _HW, _ROW)) for i in range(nP)]

    out = jnp.concatenate(outs, axis=0)
    offset = jnp.concatenate(offs, axis=0).reshape(B, 2, H, Wd)
    return out, offset


# speedup vs baseline: 1.1486x; 1.1486x over previous
import functools

import jax
import jax.numpy as jnp
from jax import lax
from jax.experimental import pallas as pl
from jax.experimental.pallas import tpu as pltpu
from jax.experimental.pallas import tpu_sc as plsc

EPS = 1e-05

_B, _C, _H, _W = 8, 96, 224, 224
_HW = _H * _W
_ROW = 128
_BLK = 3584
_NB = _HW // _BLK

_NC, _NS = 2, 16
_NCHUNK = 4
_DCH = _HW // _NCHUNK
_DUMP = _DCH
_ACC_ROWS = _DCH + 16
_SCAN = 3200
_SCAN_LO = 3072
_K = 128
_KH = 64
_NHG = _SCAN // _KH
_TSLICE = _DCH // _NS
_ZR = 16


def _stage1_body(x_ref, w_ref, b_ref, xwT_ref, dest_ref, off_ref):
    jb = pl.program_id(1)
    x_blk = x_ref[0]
    oa = lax.dot(w_ref[...], x_blk,
                 preferred_element_type=jnp.float32) + b_ref[...]
    off_y = oa[0:1] * float(_H)
    off_x = oa[1:2] * float(_W)
    att = jnp.exp(oa[2:3])
    p = jb * _BLK + lax.broadcasted_iota(jnp.int32, (1, _BLK), 1)
    dy = (p // _W).astype(jnp.float32)
    dx = (p % _W).astype(jnp.float32)
    dest_y = jnp.clip(jnp.round(dy + off_y).astype(jnp.int32), 0, _H - 1)
    dest_x = jnp.clip(jnp.round(dx + off_x).astype(jnp.int32), 0, _W - 1)
    dest = dest_y * _W + dest_x
    xw_t = (x_blk * att).T
    att_t = att.T
    pad = jnp.zeros((_BLK, _ROW - _C - 1), jnp.float32)
    xwT_ref[0] = jnp.concatenate([xw_t, att_t, pad], axis=1)
    dest_ref[0, 0] = dest[0]
    off_ref[0] = jnp.concatenate([off_y, off_x], axis=0)


@functools.lru_cache(maxsize=1)
def _make_sc_scatter():
    mesh = plsc.VectorSubcoreMesh(core_axis_name="c", subcore_axis_name="s")
    return functools.partial(
        pl.kernel,
        mesh=mesh,
        out_type=jax.ShapeDtypeStruct((_HW, _ROW), jnp.float32),
        scratch_types=[
            pltpu.VMEM((_SCAN,), jnp.int32),
            pltpu.VMEM((_NHG + 2, _KH), jnp.int32),
            pltpu.VMEM((_KH, _ROW), jnp.float32),
            pltpu.VMEM((_KH, _ROW), jnp.float32),
            pltpu.VMEM((_ZR, _ROW), jnp.float32),
            pltpu.VMEM_SHARED((_ACC_ROWS, _ROW), jnp.float32),
            pltpu.SemaphoreType.DMA,
            pltpu.SemaphoreType.DMA,
        ],
    )(_sc_scatter_body)


def _sc_scatter_body(xwT_hbm, dest_hbm, outraw_hbm,
                     destbuf, dst2d, bufA, bufB, zbuf, acc, semA, semB):
    cid = lax.axis_index("c")
    sid = lax.axis_index("s")
    zero16 = jnp.zeros((16,), jnp.float32)

    def zb_body(r, carry):
        for c in range(_ROW // 16):
            zbuf[r, pl.ds(c * 16, 16)] = zero16
        return carry

    lax.fori_loop(0, _ZR, zb_body, 0)

    base = sid * _TSLICE
    sbase = pl.multiple_of(
        sid * _SCAN_LO + jnp.minimum(sid, 8) * _K, _K)
    nhg = jnp.where(sid < 8, _SCAN // _KH, _SCAN_LO // _KH)
    lo0 = cid * 2 * _DCH
    lo1 = (cid * 2 + 1) * _DCH

    def batch_body(b, carry):
        pltpu.sync_copy(dest_hbm.at[pl.ds(sbase, _SCAN)],
                        destbuf)
        for kk, lo in ((0, lo0), (1, lo1)):
            for z in range(_TSLICE // _ZR):
                pltpu.sync_copy(zbuf, acc.at[pl.ds(base + z * _ZR, _ZR)])
            plsc.subcore_barrier()

            def build(r, carry2):
                for u in range(_KH // 16):
                    off = pl.multiple_of(r * _KH + u * 16, 16)
                    d16 = destbuf[pl.ds(off, 16)]
                    m = (d16 >= lo) & (d16 < lo + _DCH)
                    dst2d[r, pl.ds(u * 16, 16)] = jnp.where(
                        m, d16 - lo, _DUMP + sid)
                return carry2

            lax.fori_loop(0, nhg, build, 0)

            def gissue(hg, buf, sem):
                row0 = pl.multiple_of(sbase + hg * _KH, _KH)
                return pltpu.async_copy(
                    xwT_hbm.at[pl.ds(row0, _KH)], buf, sem)

            def gwait(hg, buf, sem):
                row0 = pl.multiple_of(sbase + hg * _KH, _KH)
                pltpu.make_async_copy(
                    xwT_hbm.at[pl.ds(row0, _KH)], buf, sem).wait()

            gissue(0, bufA, semA)

            def gs_body(j2, carry2):
                hga = j2 * 2
                hgb = hga + 1
                gwait(hga, bufA, semA)
                gissue(hgb, bufB, semB)
                pltpu.sync_copy(bufA, acc.at[dst2d.at[hga]], add=True)
                gwait(hgb, bufB, semB)

                @pl.when(hgb + 1 < nhg)
                def _():
                    gissue(hgb + 1, bufA, semA)

                pltpu.sync_copy(bufB, acc.at[dst2d.at[hgb]], add=True)
                return carry2

            lax.fori_loop(0, nhg // 2, gs_body, 0)
            plsc.subcore_barrier()
            pltpu.sync_copy(acc.at[pl.ds(base, _TSLICE)],
                            outraw_hbm.at[pl.ds(lo + base, _TSLICE)])
        return carry

    lax.fori_loop(0, 1, batch_body, 0)


def _stage3_body(raw_ref, out_ref):
    raw = raw_ref[0]
    fa = raw[:, 0:_C]
    aa = raw[:, _C:_C + 1] + EPS
    out_ref[0] = (fa / aa).T


def kernel(x, W, b):
    B, C, H, Wd = x.shape
    xf = x.reshape(B, C, _HW)
    sc_call = _make_sc_scatter()
    wr = W
    br = b.reshape(3, 1)

    stage1 = pl.pallas_call(
        _stage1_body,
        grid=(1, _NB),
        in_specs=[
            pl.BlockSpec((1, C, _BLK), lambda i, j: (i, 0, j)),
            pl.BlockSpec((3, C), lambda i, j: (0, 0)),
            pl.BlockSpec((3, 1), lambda i, j: (0, 0)),
        ],
        out_specs=[
            pl.BlockSpec((1, _BLK, _ROW), lambda i, j: (i, j, 0)),
            pl.BlockSpec((1, 1, _BLK), lambda i, j: (i, 0, j)),
            pl.BlockSpec((1, 2, _BLK), lambda i, j: (i, 0, j)),
        ],
        out_shape=[
            jax.ShapeDtypeStruct((1, _HW, _ROW), jnp.float32),
            jax.ShapeDtypeStruct((1, 1, _HW), jnp.int32),
            jax.ShapeDtypeStruct((1, 2, _HW), jnp.float32),
        ],
    )
    stage3 = pl.pallas_call(
        _stage3_body,
        grid=(1, _NB),
        in_specs=[pl.BlockSpec((1, _BLK, _ROW), lambda i, j: (i, j, 0))],
        out_specs=pl.BlockSpec((1, C, _BLK), lambda i, j: (i, 0, j)),
        out_shape=jax.ShapeDtypeStruct((1, C, _HW), jnp.float32),
    )

    xwTs, dests, offs = [], [], []
    for bi in range(B):
        xwT, dest, off = stage1(xf[bi:bi + 1], wr, br)
        xwTs.append(xwT.reshape(_HW, _ROW))
        dests.append(jnp.pad(dest.reshape(_HW), (0, _K),
                             constant_values=-1))
        offs.append(off)
    raws = [sc_call(xwTs[bi], dests[bi]) for bi in range(B)]
    outs = [stage3(raws[bi].reshape(1, _HW, _ROW)) for bi in range(B)]

    out = jnp.concatenate(outs, axis=0)
    offset = jnp.concatenate(offs, axis=0).reshape(B, 2, H, Wd)
    return out, offset
